# baseline (speedup 1.0000x reference)
import functools

import jax
import jax.numpy as jnp
from jax import lax
from jax.experimental import pallas as pl
from jax.experimental.pallas import tpu as pltpu
from jax.experimental.pallas import tpu_sc as plsc

_N = 10000
_E = 320000
_D = 128
_G = 64
_EPS = 1e-5

_NC = 2
_NS = 16
_NW = _NC * _NS
_CH = 128
_BPW = 80
_EPAD = _NW * _BPW * _CH
_NP = 10240
_RPT = _NP // _NS
_HB = 40


def _sc_scatter_body(h_hbm, edges_hbm, zeros_hbm, out_hbm,
                     idx0, idx1, rows0, rows1, acc, sem0, sem1):
    c = lax.axis_index("c")
    s = lax.axis_index("s")
    wid = c * _NS + s
    r0 = s * _RPT
    bstart = wid * _BPW

    pltpu.sync_copy(zeros_hbm.at[pl.ds(r0, _RPT), :], acc.at[pl.ds(r0, _RPT), :])
    plsc.subcore_barrier()

    pltpu.sync_copy(edges_hbm.at[bstart], idx0)
    pltpu.async_copy(h_hbm.at[idx0.at[0]], rows0, sem0)
    pltpu.sync_copy(edges_hbm.at[bstart + 1], idx1)
    pltpu.async_copy(h_hbm.at[idx1.at[0]], rows1, sem1)

    def body(k, carry):
        i0 = bstart + 2 * k
        pltpu.make_async_copy(h_hbm.at[idx0.at[0]], rows0, sem0).wait()
        pltpu.sync_copy(rows0, acc.at[idx0.at[1]], add=True)
        pltpu.sync_copy(edges_hbm.at[i0 + 2], idx0)
        pltpu.async_copy(h_hbm.at[idx0.at[0]], rows0, sem0)

        pltpu.make_async_copy(h_hbm.at[idx1.at[0]], rows1, sem1).wait()
        pltpu.sync_copy(rows1, acc.at[idx1.at[1]], add=True)
        pltpu.sync_copy(edges_hbm.at[i0 + 3], idx1)
        pltpu.async_copy(h_hbm.at[idx1.at[0]], rows1, sem1)
        return carry

    lax.fori_loop(0, _BPW // 2 - 1, body, 0)

    pltpu.make_async_copy(h_hbm.at[idx0.at[0]], rows0, sem0).wait()
    pltpu.sync_copy(rows0, acc.at[idx0.at[1]], add=True)
    pltpu.make_async_copy(h_hbm.at[idx1.at[0]], rows1, sem1).wait()
    pltpu.sync_copy(rows1, acc.at[idx1.at[1]], add=True)

    plsc.subcore_barrier()
    pltpu.sync_copy(acc.at[pl.ds(r0, _RPT), :], out_hbm.at[c, pl.ds(r0, _RPT), :])


@functools.cache
def _get_sc_scatter():
    return pl.kernel(
        _sc_scatter_body,
        out_type=jax.ShapeDtypeStruct((_NC, _NP, _D), jnp.float32),
        mesh=plsc.VectorSubcoreMesh(core_axis_name="c", subcore_axis_name="s"),
        scratch_types=[
            pltpu.VMEM((2, _CH), jnp.int32),
            pltpu.VMEM((2, _CH), jnp.int32),
            pltpu.VMEM((_CH, _D), jnp.float32),
            pltpu.VMEM((_CH, _D), jnp.float32),
            pltpu.VMEM_SHARED((_NP, _D), jnp.float32),
            pltpu.SemaphoreType.DMA,
            pltpu.SemaphoreType.DMA,
        ],
    )


def _tc_layer_body(final, h_ref, aggr_ref, batch_ref,
                   vw_ref, vb_ref, aw_ref, ab_ref, rw_ref, rb_ref,
                   g_ref, b_ref, lw_ref, lb_ref, out_ref):
    h = h_ref[...]
    aggr = (aggr_ref[0] + aggr_ref[1])[:_N]
    onehot = (batch_ref[...] ==
              lax.broadcasted_iota(jnp.int32, (_N, _G), 1)).astype(jnp.float32)
    pooled = lax.dot_general(onehot, h, (((0,), (0,)), ((), ())),
                             preferred_element_type=jnp.float32)
    pr = jnp.dot(pooled, rw_ref[...], preferred_element_type=jnp.float32)
    comb = (jnp.dot(h, vw_ref[...], preferred_element_type=jnp.float32)
            + jnp.dot(aggr, aw_ref[...], preferred_element_type=jnp.float32)
            + jnp.dot(onehot, pr, preferred_element_type=jnp.float32)
            + vb_ref[...] + ab_ref[...] + rb_ref[...])
    hr = jnp.maximum(comb, 0.0)
    mean = jnp.mean(hr, axis=0, keepdims=True)
    var = jnp.mean((hr - mean) * (hr - mean), axis=0, keepdims=True)
    hn = (hr - mean) * lax.rsqrt(var + _EPS) * g_ref[...] + b_ref[...]
    if final:
        out_ref[...] = (jnp.dot(hn, lw_ref[...],
                                preferred_element_type=jnp.float32)
                        + lb_ref[...])
    else:
        out_ref[...] = hn


def _tc_layer(final, h, aggr, batch_col, vw, vb, aw, ab, rw, rb, g, b, lw, lb):
    return pl.pallas_call(
        functools.partial(_tc_layer_body, final),
        out_shape=jax.ShapeDtypeStruct((_N, lw.shape[1] if final else _D),
                                       jnp.float32),
    )(h, aggr, batch_col, vw, vb.reshape(1, -1), aw, ab.reshape(1, -1),
      rw, rb.reshape(1, -1), g.reshape(1, -1), b.reshape(1, -1),
      lw, lb.reshape(1, -1))


def kernel(x, edge_index, batch,
           V0w, V0b, A0w, A0b, R0w, R0b, bn0_g, bn0_b,
           V1w, V1b, A1w, A1b, R1w, R1b, bn1_g, bn1_b,
           lin_w, lin_b):
    npad = _EPAD - _E
    src = jnp.concatenate(
        [edge_index[0], jnp.zeros((npad,), jnp.int32)]).reshape(-1, 1, _CH)
    dst = jnp.concatenate(
        [edge_index[1], jnp.full((npad,), _N, jnp.int32)]).reshape(-1, 1, _CH)
    edges = jnp.concatenate([src, dst], axis=1)
    zeros = jnp.zeros((_NP, _D), jnp.float32)
    batch_col = batch.reshape(_N, 1)

    sc_scatter = _get_sc_scatter()
    aggr0 = sc_scatter(x, edges, zeros)
    h1 = _tc_layer(False, x, aggr0, batch_col,
                   V0w, V0b, A0w, A0b, R0w, R0b, bn0_g, bn0_b, lin_w, lin_b)
    aggr1 = sc_scatter(h1, edges, zeros)
    out = _tc_layer(True, h1, aggr1, batch_col,
                    V1w, V1b, A1w, A1b, R1w, R1b, bn1_g, bn1_b, lin_w, lin_b)
    return out


# speedup vs baseline: 1.0001x; 1.0001x over previous
import functools

import jax
import jax.numpy as jnp
from jax import lax
from jax.experimental import pallas as pl
from jax.experimental.pallas import tpu as pltpu
from jax.experimental.pallas import tpu_sc as plsc

_N = 10000
_E = 320000
_D = 128
_G = 64
_EPS = 1e-5

_NC = 2
_NS = 16
_NW = _NC * _NS
_CH = 128
_BPW = 80
_EPAD = _NW * _BPW * _CH
_NP = 10240
_RPT = _NP // _NS
_HB = 40


def _sc_scatter_body(h_hbm, edges_hbm, zeros_hbm, out_hbm,
                     idx0, idx1, rows0, rows1, acc, sem0, sem1):
    c = lax.axis_index("c")
    s = lax.axis_index("s")
    wid = c * _NS + s
    r0 = s * _RPT
    bstart = wid * _BPW

    with jax.named_scope("sc_zero"):
        pltpu.sync_copy(zeros_hbm.at[pl.ds(r0, _RPT), :],
                        acc.at[pl.ds(r0, _RPT), :])
        plsc.subcore_barrier()

    with jax.named_scope("sc_prologue"):
        pltpu.sync_copy(edges_hbm.at[bstart], idx0)
        pltpu.async_copy(h_hbm.at[idx0.at[0]], rows0, sem0)
        pltpu.sync_copy(edges_hbm.at[bstart + 1], idx1)
        pltpu.async_copy(h_hbm.at[idx1.at[0]], rows1, sem1)

    def body(k, carry):
        i0 = bstart + 2 * k
        with jax.named_scope("sc_wait0"):
            pltpu.make_async_copy(h_hbm.at[idx0.at[0]], rows0, sem0).wait()
        with jax.named_scope("sc_scat0"):
            pltpu.sync_copy(rows0, acc.at[idx0.at[1]], add=True)
        with jax.named_scope("sc_idx0"):
            pltpu.sync_copy(edges_hbm.at[i0 + 2], idx0)
        with jax.named_scope("sc_gath0"):
            pltpu.async_copy(h_hbm.at[idx0.at[0]], rows0, sem0)

        with jax.named_scope("sc_wait1"):
            pltpu.make_async_copy(h_hbm.at[idx1.at[0]], rows1, sem1).wait()
        with jax.named_scope("sc_scat1"):
            pltpu.sync_copy(rows1, acc.at[idx1.at[1]], add=True)
        with jax.named_scope("sc_idx1"):
            pltpu.sync_copy(edges_hbm.at[i0 + 3], idx1)
        with jax.named_scope("sc_gath1"):
            pltpu.async_copy(h_hbm.at[idx1.at[0]], rows1, sem1)
        return carry

    lax.fori_loop(0, _BPW // 2 - 1, body, 0)

    with jax.named_scope("sc_drain"):
        pltpu.make_async_copy(h_hbm.at[idx0.at[0]], rows0, sem0).wait()
        pltpu.sync_copy(rows0, acc.at[idx0.at[1]], add=True)
        pltpu.make_async_copy(h_hbm.at[idx1.at[0]], rows1, sem1).wait()
        pltpu.sync_copy(rows1, acc.at[idx1.at[1]], add=True)

    plsc.subcore_barrier()
    pltpu.sync_copy(acc.at[pl.ds(r0, _RPT), :], out_hbm.at[c, pl.ds(r0, _RPT), :])


@functools.cache
def _get_sc_scatter():
    return pl.kernel(
        _sc_scatter_body,
        out_type=jax.ShapeDtypeStruct((_NC, _NP, _D), jnp.float32),
        mesh=plsc.VectorSubcoreMesh(core_axis_name="c", subcore_axis_name="s"),
        scratch_types=[
            pltpu.VMEM((2, _CH), jnp.int32),
            pltpu.VMEM((2, _CH), jnp.int32),
            pltpu.VMEM((_CH, _D), jnp.float32),
            pltpu.VMEM((_CH, _D), jnp.float32),
            pltpu.VMEM_SHARED((_NP, _D), jnp.float32),
            pltpu.SemaphoreType.DMA,
            pltpu.SemaphoreType.DMA,
        ],
    )


def _tc_layer_body(final, h_ref, aggr_ref, batch_ref,
                   vw_ref, vb_ref, aw_ref, ab_ref, rw_ref, rb_ref,
                   g_ref, b_ref, lw_ref, lb_ref, out_ref):
    h = h_ref[...]
    aggr = (aggr_ref[0] + aggr_ref[1])[:_N]
    onehot = (batch_ref[...] ==
              lax.broadcasted_iota(jnp.int32, (_N, _G), 1)).astype(jnp.float32)
    pooled = lax.dot_general(onehot, h, (((0,), (0,)), ((), ())),
                             preferred_element_type=jnp.float32)
    pr = jnp.dot(pooled, rw_ref[...], preferred_element_type=jnp.float32)
    comb = (jnp.dot(h, vw_ref[...], preferred_element_type=jnp.float32)
            + jnp.dot(aggr, aw_ref[...], preferred_element_type=jnp.float32)
            + jnp.dot(onehot, pr, preferred_element_type=jnp.float32)
            + vb_ref[...] + ab_ref[...] + rb_ref[...])
    hr = jnp.maximum(comb, 0.0)
    mean = jnp.mean(hr, axis=0, keepdims=True)
    var = jnp.mean((hr - mean) * (hr - mean), axis=0, keepdims=True)
    hn = (hr - mean) * lax.rsqrt(var + _EPS) * g_ref[...] + b_ref[...]
    if final:
        out_ref[...] = (jnp.dot(hn, lw_ref[...],
                                preferred_element_type=jnp.float32)
                        + lb_ref[...])
    else:
        out_ref[...] = hn


def _tc_layer(final, h, aggr, batch_col, vw, vb, aw, ab, rw, rb, g, b, lw, lb):
    return pl.pallas_call(
        functools.partial(_tc_layer_body, final),
        out_shape=jax.ShapeDtypeStruct((_N, lw.shape[1] if final else _D),
                                       jnp.float32),
    )(h, aggr, batch_col, vw, vb.reshape(1, -1), aw, ab.reshape(1, -1),
      rw, rb.reshape(1, -1), g.reshape(1, -1), b.reshape(1, -1),
      lw, lb.reshape(1, -1))


def kernel(x, edge_index, batch,
           V0w, V0b, A0w, A0b, R0w, R0b, bn0_g, bn0_b,
           V1w, V1b, A1w, A1b, R1w, R1b, bn1_g, bn1_b,
           lin_w, lin_b):
    npad = _EPAD - _E
    src = jnp.concatenate(
        [edge_index[0], jnp.zeros((npad,), jnp.int32)]).reshape(-1, 1, _CH)
    dst = jnp.concatenate(
        [edge_index[1], jnp.full((npad,), _N, jnp.int32)]).reshape(-1, 1, _CH)
    edges = jnp.concatenate([src, dst], axis=1)
    zeros = jnp.zeros((_NP, _D), jnp.float32)
    batch_col = batch.reshape(_N, 1)

    sc_scatter = _get_sc_scatter()
    aggr0 = sc_scatter(x, edges, zeros)
    h1 = _tc_layer(False, x, aggr0, batch_col,
                   V0w, V0b, A0w, A0b, R0w, R0b, bn0_g, bn0_b, lin_w, lin_b)
    aggr1 = sc_scatter(h1, edges, zeros)
    out = _tc_layer(True, h1, aggr1, batch_col,
                    V1w, V1b, A1w, A1b, R1w, R1b, bn1_g, bn1_b, lin_w, lin_b)
    return out


# speedup vs baseline: 1.2090x; 1.2089x over previous
import functools

import jax
import jax.numpy as jnp
from jax import lax
from jax.experimental import pallas as pl
from jax.experimental.pallas import tpu as pltpu
from jax.experimental.pallas import tpu_sc as plsc

_N = 10000
_E = 320000
_D = 128
_H = 64
_G = 64
_EPS = 1e-5

_NC = 2
_NS = 16
_CH = 128
_KSB = 8
_NB = 2560
_EPAD = _NB * _CH
_NSB = _NB // _KSB
_SBT = _NSB // _NS
_NP = 10240
_RPT = _NP // _NS


def _sc_scatter_body(h0_hbm, h1_hbm, edges_hbm, zeros_hbm, out_hbm,
                     idx, rows, acc, sem):
    c = lax.axis_index("c")
    s = lax.axis_index("s")
    r0 = s * _RPT
    sb0 = s * _SBT

    with jax.named_scope("sc_zero"):
        pltpu.sync_copy(zeros_hbm.at[pl.ds(r0, _RPT), :],
                        acc.at[pl.ds(r0, _RPT), :])
        plsc.subcore_barrier()

    for core, h_hbm in ((0, h0_hbm), (1, h1_hbm)):
        @pl.when(c == core)
        def _(h_hbm=h_hbm):
            def body(j, carry):
                sb = sb0 + j
                with jax.named_scope("sc_idx"):
                    pltpu.sync_copy(edges_hbm.at[sb], idx)
                with jax.named_scope("sc_gath"):
                    pltpu.async_copy(h_hbm.at[idx.at[0]], rows, sem).wait()
                with jax.named_scope("sc_scat"):
                    pltpu.sync_copy(rows, acc.at[idx.at[1]], add=True)
                return carry

            lax.fori_loop(0, _SBT, body, 0)

    plsc.subcore_barrier()
    with jax.named_scope("sc_out"):
        pltpu.sync_copy(acc.at[pl.ds(r0, _RPT), :],
                        out_hbm.at[c, pl.ds(r0, _RPT), :])


@functools.cache
def _get_sc_scatter():
    return pl.kernel(
        _sc_scatter_body,
        out_type=jax.ShapeDtypeStruct((_NC, _NP, _H), jnp.float32),
        mesh=plsc.VectorSubcoreMesh(core_axis_name="c", subcore_axis_name="s"),
        compiler_params=pltpu.CompilerParams(use_tc_tiling_on_sc=False),
        scratch_types=[
            pltpu.VMEM((2, _KSB * _CH), jnp.int32),
            pltpu.VMEM((_KSB * _CH, _H), jnp.float32),
            pltpu.VMEM_SHARED((_NP, _H), jnp.float32),
            pltpu.SemaphoreType.DMA,
        ],
    )


def _tc_layer_body(final, ha_ref, hb_ref, aggr_ref, batch_ref,
                   vw_ref, vb_ref, aw_ref, ab_ref, rw_ref, rb_ref,
                   g_ref, b_ref, lw_ref, lb_ref, *out_refs):
    h = jnp.concatenate([ha_ref[...], hb_ref[...]], axis=1)
    aggr = jnp.concatenate([aggr_ref[0, :_N], aggr_ref[1, :_N]], axis=1)
    onehot = (batch_ref[...] ==
              lax.broadcasted_iota(jnp.int32, (_N, _G), 1)).astype(jnp.float32)
    pooled = lax.dot_general(onehot, h, (((0,), (0,)), ((), ())),
                             preferred_element_type=jnp.float32)
    pr = jnp.dot(pooled, rw_ref[...], preferred_element_type=jnp.float32)
    comb = (jnp.dot(h, vw_ref[...], preferred_element_type=jnp.float32)
            + jnp.dot(aggr, aw_ref[...], preferred_element_type=jnp.float32)
            + jnp.dot(onehot, pr, preferred_element_type=jnp.float32)
            + vb_ref[...] + ab_ref[...] + rb_ref[...])
    hr = jnp.maximum(comb, 0.0)
    mean = jnp.mean(hr, axis=0, keepdims=True)
    var = jnp.mean((hr - mean) * (hr - mean), axis=0, keepdims=True)
    hn = (hr - mean) * lax.rsqrt(var + _EPS) * g_ref[...] + b_ref[...]
    if final:
        out_refs[0][...] = (jnp.dot(hn, lw_ref[...],
                                    preferred_element_type=jnp.float32)
                            + lb_ref[...])
    else:
        out_refs[0][...] = hn[:, :_H]
        out_refs[1][...] = hn[:, _H:]


def _tc_layer(final, ha, hb, aggr, batch_col,
              vw, vb, aw, ab, rw, rb, g, b, lw, lb):
    if final:
        out_shape = jax.ShapeDtypeStruct((_N, lw.shape[1]), jnp.float32)
    else:
        out_shape = (jax.ShapeDtypeStruct((_N, _H), jnp.float32),
                     jax.ShapeDtypeStruct((_N, _H), jnp.float32))
    return pl.pallas_call(
        functools.partial(_tc_layer_body, final),
        out_shape=out_shape,
    )(ha, hb, aggr, batch_col, vw, vb.reshape(1, -1), aw, ab.reshape(1, -1),
      rw, rb.reshape(1, -1), g.reshape(1, -1), b.reshape(1, -1),
      lw, lb.reshape(1, -1))


def kernel(x, edge_index, batch,
           V0w, V0b, A0w, A0b, R0w, R0b, bn0_g, bn0_b,
           V1w, V1b, A1w, A1b, R1w, R1b, bn1_g, bn1_b,
           lin_w, lin_b):
    npad = _EPAD - _E
    src = jnp.concatenate(
        [edge_index[0], jnp.zeros((npad,), jnp.int32)]).reshape(_NSB, 1, _KSB * _CH)
    dst = jnp.concatenate(
        [edge_index[1], jnp.full((npad,), _N, jnp.int32)]).reshape(_NSB, 1, _KSB * _CH)
    edges = jnp.concatenate([src, dst], axis=1)
    zeros = jnp.zeros((_NP, _H), jnp.float32)
    batch_col = batch.reshape(_N, 1)
    xa = x[:, :_H]
    xb = x[:, _H:]

    sc_scatter = _get_sc_scatter()
    aggr0 = sc_scatter(xa, xb, edges, zeros)
    h1a, h1b = _tc_layer(False, xa, xb, aggr0, batch_col,
                         V0w, V0b, A0w, A0b, R0w, R0b, bn0_g, bn0_b,
                         lin_w, lin_b)
    aggr1 = sc_scatter(h1a, h1b, edges, zeros)
    out = _tc_layer(True, h1a, h1b, aggr1, batch_col,
                    V1w, V1b, A1w, A1b, R1w, R1b, bn1_g, bn1_b, lin_w, lin_b)
    return out


# speedup vs baseline: 1.3927x; 1.1520x over previous
import functools

import jax
import jax.numpy as jnp
from jax import lax
from jax.experimental import pallas as pl
from jax.experimental.pallas import tpu as pltpu
from jax.experimental.pallas import tpu_sc as plsc

_N = 10000
_E = 320000
_D = 128
_H = 64
_G = 64
_EPS = 1e-5

_NC = 2
_NS = 16
_CH = 128
_KSB = 8
_NB = 2560
_EPAD = _NB * _CH
_NSB = _NB // _KSB
_SBT = _NSB // _NS
_NP = 10240
_RPT = _NP // _NS


def _sc_scatter_body(h0_hbm, h1_hbm, edges_hbm, zeros_hbm, out_hbm,
                     idx, rows, acc, sem):
    c = lax.axis_index("c")
    s = lax.axis_index("s")
    r0 = s * _RPT

    with jax.named_scope("sc_zero"):
        pltpu.sync_copy(zeros_hbm.at[pl.ds(r0, _RPT), :],
                        acc.at[pl.ds(r0, _RPT), :])
        plsc.subcore_barrier()

    for core, h_hbm in ((0, h0_hbm), (1, h1_hbm)):
        @pl.when(c == core)
        def _(h_hbm=h_hbm):
            def body(j, carry):
                sb = s + j * _NS
                with jax.named_scope("sc_idx"):
                    pltpu.sync_copy(edges_hbm.at[sb], idx)
                with jax.named_scope("sc_gath"):
                    pltpu.async_copy(h_hbm.at[idx.at[0]], rows, sem).wait()
                with jax.named_scope("sc_scat"):
                    pltpu.sync_copy(rows, acc.at[idx.at[1]], add=True)
                return carry

            lax.fori_loop(0, _SBT, body, 0)

    plsc.subcore_barrier()
    with jax.named_scope("sc_out"):
        pltpu.sync_copy(acc.at[pl.ds(r0, _RPT), :],
                        out_hbm.at[c, pl.ds(r0, _RPT), :])


@functools.cache
def _get_sc_scatter():
    return pl.kernel(
        _sc_scatter_body,
        out_type=jax.ShapeDtypeStruct((_NC, _NP, _H), jnp.float32),
        mesh=plsc.VectorSubcoreMesh(core_axis_name="c", subcore_axis_name="s"),
        compiler_params=pltpu.CompilerParams(use_tc_tiling_on_sc=False),
        scratch_types=[
            pltpu.VMEM((2, _KSB * _CH), jnp.int32),
            pltpu.VMEM((_KSB * _CH, _H), jnp.float32),
            pltpu.VMEM_SHARED((_NP, _H), jnp.float32),
            pltpu.SemaphoreType.DMA,
        ],
    )


def _tc_layer_body(final, ha_ref, hb_ref, aggr_ref, batch_ref,
                   vw_ref, vb_ref, aw_ref, ab_ref, rw_ref, rb_ref,
                   g_ref, b_ref, lw_ref, lb_ref, *out_refs):
    h = jnp.concatenate([ha_ref[...], hb_ref[...]], axis=1)
    aggr = jnp.concatenate([aggr_ref[0, :_N], aggr_ref[1, :_N]], axis=1)
    onehot = (batch_ref[...] ==
              lax.broadcasted_iota(jnp.int32, (_N, _G), 1)).astype(jnp.float32)
    pooled = lax.dot_general(onehot, h, (((0,), (0,)), ((), ())),
                             preferred_element_type=jnp.float32)
    pr = jnp.dot(pooled, rw_ref[...], preferred_element_type=jnp.float32)
    comb = (jnp.dot(h, vw_ref[...], preferred_element_type=jnp.float32)
            + jnp.dot(aggr, aw_ref[...], preferred_element_type=jnp.float32)
            + jnp.dot(onehot, pr, preferred_element_type=jnp.float32)
            + vb_ref[...] + ab_ref[...] + rb_ref[...])
    hr = jnp.maximum(comb, 0.0)
    mean = jnp.mean(hr, axis=0, keepdims=True)
    var = jnp.mean((hr - mean) * (hr - mean), axis=0, keepdims=True)
    hn = (hr - mean) * lax.rsqrt(var + _EPS) * g_ref[...] + b_ref[...]
    if final:
        out_refs[0][...] = (jnp.dot(hn, lw_ref[...],
                                    preferred_element_type=jnp.float32)
                            + lb_ref[...])
    else:
        out_refs[0][...] = hn[:, :_H]
        out_refs[1][...] = hn[:, _H:]


def _tc_layer(final, ha, hb, aggr, batch_col,
              vw, vb, aw, ab, rw, rb, g, b, lw, lb):
    if final:
        out_shape = jax.ShapeDtypeStruct((_N, lw.shape[1]), jnp.float32)
    else:
        out_shape = (jax.ShapeDtypeStruct((_N, _H), jnp.float32),
                     jax.ShapeDtypeStruct((_N, _H), jnp.float32))
    return pl.pallas_call(
        functools.partial(_tc_layer_body, final),
        out_shape=out_shape,
    )(ha, hb, aggr, batch_col, vw, vb.reshape(1, -1), aw, ab.reshape(1, -1),
      rw, rb.reshape(1, -1), g.reshape(1, -1), b.reshape(1, -1),
      lw, lb.reshape(1, -1))


def kernel(x, edge_index, batch,
           V0w, V0b, A0w, A0b, R0w, R0b, bn0_g, bn0_b,
           V1w, V1b, A1w, A1b, R1w, R1b, bn1_g, bn1_b,
           lin_w, lin_b):
    npad = _EPAD - _E
    src = jnp.concatenate(
        [edge_index[0], jnp.zeros((npad,), jnp.int32)]).reshape(_NSB, 1, _KSB * _CH)
    pad_dst = _N + jnp.arange(npad, dtype=jnp.int32) % (_NP - _N)
    dst = jnp.concatenate(
        [edge_index[1], pad_dst]).reshape(_NSB, 1, _KSB * _CH)
    edges = jnp.concatenate([src, dst], axis=1)
    zeros = jnp.zeros((_NP, _H), jnp.float32)
    batch_col = batch.reshape(_N, 1)
    xa = x[:, :_H]
    xb = x[:, _H:]

    sc_scatter = _get_sc_scatter()
    aggr0 = sc_scatter(xa, xb, edges, zeros)
    h1a, h1b = _tc_layer(False, xa, xb, aggr0, batch_col,
                         V0w, V0b, A0w, A0b, R0w, R0b, bn0_g, bn0_b,
                         lin_w, lin_b)
    aggr1 = sc_scatter(h1a, h1b, edges, zeros)
    out = _tc_layer(True, h1a, h1b, aggr1, batch_col,
                    V1w, V1b, A1w, A1b, R1w, R1b, bn1_g, bn1_b, lin_w, lin_b)
    return out


# speedup vs baseline: 2.3302x; 1.6732x over previous
import functools

import jax
import jax.numpy as jnp
from jax import lax
from jax.experimental import pallas as pl
from jax.experimental.pallas import tpu as pltpu
from jax.experimental.pallas import tpu_sc as plsc

_N = 10000
_E = 320000
_D = 128
_H = 64
_G = 64
_EPS = 1e-5

_NC = 2
_NS = 16
_CH = 128
_KSB = 8
_NB = 2560
_EPAD = _NB * _CH
_NSB = _NB // _KSB
_SBT = _NSB // _NS
_NP = 10240
_RPT = _NP // _NS


def _sc_scatter_body(h0_hbm, h1_hbm, edges_hbm, zeros_hbm, out_hbm,
                     idx, rows, acc, sem):
    c = lax.axis_index("c")
    s = lax.axis_index("s")
    r0 = s * _RPT

    with jax.named_scope("sc_zero"):
        pltpu.sync_copy(zeros_hbm.at[pl.ds(r0, _RPT), :],
                        acc.at[pl.ds(r0, _RPT), :])
        plsc.subcore_barrier()

    for core, h_hbm in ((0, h0_hbm), (1, h1_hbm)):
        @pl.when(c == core)
        def _(h_hbm=h_hbm):
            def body(j, carry):
                sb = s + j * _NS
                with jax.named_scope("sc_idx"):
                    pltpu.sync_copy(edges_hbm.at[sb], idx)
                with jax.named_scope("sc_gath"):
                    pltpu.async_copy(h_hbm.at[idx.at[0]], rows, sem).wait()
                with jax.named_scope("sc_scat"):
                    pltpu.sync_copy(rows, acc.at[idx.at[1]], add=True)
                return carry

            lax.fori_loop(0, _SBT, body, 0)

    plsc.subcore_barrier()
    with jax.named_scope("sc_out"):
        pltpu.sync_copy(acc.at[pl.ds(r0, _RPT), :],
                        out_hbm.at[c, pl.ds(r0, _RPT), :])


@functools.cache
def _get_sc_scatter():
    return pl.kernel(
        _sc_scatter_body,
        out_type=jax.ShapeDtypeStruct((_NC, _NP, _H), jnp.float32),
        mesh=plsc.VectorSubcoreMesh(core_axis_name="c", subcore_axis_name="s"),
        compiler_params=pltpu.CompilerParams(use_tc_tiling_on_sc=False),
        scratch_types=[
            pltpu.VMEM((2, _KSB * _CH), jnp.int32),
            pltpu.VMEM((_KSB * _CH, _H), jnp.float32),
            pltpu.VMEM_SHARED((_NP, _H), jnp.float32),
            pltpu.SemaphoreType.DMA,
        ],
    )


def _tc_layer_body(final, ha_ref, hb_ref, aggr_ref, batch_ref,
                   vw_ref, vb_ref, aw_ref, ab_ref, rw_ref, rb_ref,
                   g_ref, b_ref, lw_ref, lb_ref, *out_refs):
    h = jnp.concatenate([ha_ref[...], hb_ref[...]], axis=1)
    aggr = jnp.concatenate([aggr_ref[0, :_N], aggr_ref[1, :_N]], axis=1)
    onehot = (batch_ref[...] ==
              lax.broadcasted_iota(jnp.int32, (_N, _G), 1)).astype(jnp.float32)
    pooled = lax.dot_general(onehot, h, (((0,), (0,)), ((), ())),
                             preferred_element_type=jnp.float32)
    pr = jnp.dot(pooled, rw_ref[...], preferred_element_type=jnp.float32)
    comb = (jnp.dot(h, vw_ref[...], preferred_element_type=jnp.float32)
            + jnp.dot(aggr, aw_ref[...], preferred_element_type=jnp.float32)
            + jnp.dot(onehot, pr, preferred_element_type=jnp.float32)
            + vb_ref[...] + ab_ref[...] + rb_ref[...])
    hr = jnp.maximum(comb, 0.0)
    mean = jnp.mean(hr, axis=0, keepdims=True)
    var = jnp.mean((hr - mean) * (hr - mean), axis=0, keepdims=True)
    hn = (hr - mean) * lax.rsqrt(var + _EPS) * g_ref[...] + b_ref[...]
    if final:
        out_refs[0][...] = (jnp.dot(hn, lw_ref[...],
                                    preferred_element_type=jnp.float32)
                            + lb_ref[...])
    else:
        out_refs[0][...] = hn[:, :_H]
        out_refs[1][...] = hn[:, _H:]


def _tc_layer(final, ha, hb, aggr, batch_col,
              vw, vb, aw, ab, rw, rb, g, b, lw, lb):
    if final:
        out_shape = jax.ShapeDtypeStruct((_N, lw.shape[1]), jnp.float32)
    else:
        out_shape = (jax.ShapeDtypeStruct((_N, _H), jnp.float32),
                     jax.ShapeDtypeStruct((_N, _H), jnp.float32))
    return pl.pallas_call(
        functools.partial(_tc_layer_body, final),
        out_shape=out_shape,
    )(ha, hb, aggr, batch_col, vw, vb.reshape(1, -1), aw, ab.reshape(1, -1),
      rw, rb.reshape(1, -1), g.reshape(1, -1), b.reshape(1, -1),
      lw, lb.reshape(1, -1))


def kernel(x, edge_index, batch,
           V0w, V0b, A0w, A0b, R0w, R0b, bn0_g, bn0_b,
           V1w, V1b, A1w, A1b, R1w, R1b, bn1_g, bn1_b,
           lin_w, lin_b):
    npad = _EPAD - _E
    pad_src = jnp.arange(npad, dtype=jnp.int32) % _N
    src = jnp.concatenate(
        [edge_index[0], pad_src]).reshape(_NSB, 1, _KSB * _CH)
    pad_dst = _N + jnp.arange(npad, dtype=jnp.int32) % (_NP - _N)
    dst = jnp.concatenate(
        [edge_index[1], pad_dst]).reshape(_NSB, 1, _KSB * _CH)
    edges = jnp.concatenate([src, dst], axis=1)
    zeros = jnp.zeros((_NP, _H), jnp.float32)
    batch_col = batch.reshape(_N, 1)
    xa = x[:, :_H]
    xb = x[:, _H:]

    sc_scatter = _get_sc_scatter()
    aggr0 = sc_scatter(xa, xb, edges, zeros)
    h1a, h1b = _tc_layer(False, xa, xb, aggr0, batch_col,
                         V0w, V0b, A0w, A0b, R0w, R0b, bn0_g, bn0_b,
                         lin_w, lin_b)
    aggr1 = sc_scatter(h1a, h1b, edges, zeros)
    out = _tc_layer(True, h1a, h1b, aggr1, batch_col,
                    V1w, V1b, A1w, A1b, R1w, R1b, bn1_g, bn1_b, lin_w, lin_b)
    return out


# speedup vs baseline: 2.8639x; 1.2290x over previous
import functools

import jax
import jax.numpy as jnp
from jax import lax
from jax.experimental import pallas as pl
from jax.experimental.pallas import tpu as pltpu
from jax.experimental.pallas import tpu_sc as plsc

_N = 10000
_E = 320000
_D = 128
_H = 64
_G = 64
_EPS = 1e-5

_NC = 2
_NS = 16
_SBE = 500
_NSB = _E // _SBE
_SBT = _NSB // _NS
_NP = 10240
_RPT = _NP // _NS


def _sc_scatter_body(h0_hbm, h1_hbm, edges_hbm, zeros_hbm, out_hbm,
                     idxa, idxb, rowsa, rowsb, acc,
                     gsema, gsemb, isema, isemb):
    c = lax.axis_index("c")
    s = lax.axis_index("s")
    r0 = s * _RPT

    with jax.named_scope("sc_zero"):
        pltpu.sync_copy(zeros_hbm.at[pl.ds(r0, _RPT), :],
                        acc.at[pl.ds(r0, _RPT), :])
        plsc.subcore_barrier()

    def sb_of(j):
        return s + j * _NS

    for core, h_hbm in ((0, h0_hbm), (1, h1_hbm)):
        @pl.when(c == core)
        def _(h_hbm=h_hbm):
            pltpu.sync_copy(edges_hbm.at[sb_of(0)], idxa)
            pltpu.async_copy(h_hbm.at[idxa.at[0]], rowsa, gsema)
            pltpu.async_copy(edges_hbm.at[sb_of(1)], idxb, isemb)

            def body(k, carry):
                j0 = 2 * k
                pltpu.make_async_copy(edges_hbm.at[sb_of(j0 + 1)],
                                      idxb, isemb).wait()
                pltpu.async_copy(h_hbm.at[idxb.at[0]], rowsb, gsemb)
                pltpu.make_async_copy(h_hbm.at[idxa.at[0]],
                                      rowsa, gsema).wait()
                with jax.named_scope("sc_scat"):
                    pltpu.sync_copy(rowsa, acc.at[idxa.at[1]], add=True)
                pltpu.async_copy(edges_hbm.at[sb_of(j0 + 2)], idxa, isema)

                pltpu.make_async_copy(edges_hbm.at[sb_of(j0 + 2)],
                                      idxa, isema).wait()
                pltpu.async_copy(h_hbm.at[idxa.at[0]], rowsa, gsema)
                pltpu.make_async_copy(h_hbm.at[idxb.at[0]],
                                      rowsb, gsemb).wait()
                with jax.named_scope("sc_scat"):
                    pltpu.sync_copy(rowsb, acc.at[idxb.at[1]], add=True)
                pltpu.async_copy(edges_hbm.at[sb_of(j0 + 3)], idxb, isemb)
                return carry

            lax.fori_loop(0, _SBT // 2 - 1, body, 0)

            pltpu.make_async_copy(edges_hbm.at[sb_of(_SBT - 1)],
                                  idxb, isemb).wait()
            pltpu.async_copy(h_hbm.at[idxb.at[0]], rowsb, gsemb)
            pltpu.make_async_copy(h_hbm.at[idxa.at[0]], rowsa, gsema).wait()
            with jax.named_scope("sc_scat"):
                pltpu.sync_copy(rowsa, acc.at[idxa.at[1]], add=True)
            pltpu.make_async_copy(h_hbm.at[idxb.at[0]], rowsb, gsemb).wait()
            with jax.named_scope("sc_scat"):
                pltpu.sync_copy(rowsb, acc.at[idxb.at[1]], add=True)

    plsc.subcore_barrier()
    with jax.named_scope("sc_out"):
        pltpu.sync_copy(acc.at[pl.ds(r0, _RPT), :],
                        out_hbm.at[c, pl.ds(r0, _RPT), :])


@functools.cache
def _get_sc_scatter():
    return pl.kernel(
        _sc_scatter_body,
        out_type=jax.ShapeDtypeStruct((_NC, _NP, _H), jnp.float32),
        mesh=plsc.VectorSubcoreMesh(core_axis_name="c", subcore_axis_name="s"),
        compiler_params=pltpu.CompilerParams(use_tc_tiling_on_sc=False),
        scratch_types=[
            pltpu.VMEM((2, _SBE), jnp.int32),
            pltpu.VMEM((2, _SBE), jnp.int32),
            pltpu.VMEM((_SBE, _H), jnp.float32),
            pltpu.VMEM((_SBE, _H), jnp.float32),
            pltpu.VMEM_SHARED((_NP, _H), jnp.float32),
            pltpu.SemaphoreType.DMA,
            pltpu.SemaphoreType.DMA,
            pltpu.SemaphoreType.DMA,
            pltpu.SemaphoreType.DMA,
        ],
    )


def _tc_layer_body(final, ha_ref, hb_ref, aggr_ref, batch_ref,
                   vw_ref, vb_ref, aw_ref, ab_ref, rw_ref, rb_ref,
                   g_ref, b_ref, lw_ref, lb_ref, *out_refs):
    h = jnp.concatenate([ha_ref[...], hb_ref[...]], axis=1)
    aggr = jnp.concatenate([aggr_ref[0, :_N], aggr_ref[1, :_N]], axis=1)
    onehot = (batch_ref[...] ==
              lax.broadcasted_iota(jnp.int32, (_N, _G), 1)).astype(jnp.float32)
    pooled = lax.dot_general(onehot, h, (((0,), (0,)), ((), ())),
                             preferred_element_type=jnp.float32)
    pr = jnp.dot(pooled, rw_ref[...], preferred_element_type=jnp.float32)
    comb = (jnp.dot(h, vw_ref[...], preferred_element_type=jnp.float32)
            + jnp.dot(aggr, aw_ref[...], preferred_element_type=jnp.float32)
            + jnp.dot(onehot, pr, preferred_element_type=jnp.float32)
            + vb_ref[...] + ab_ref[...] + rb_ref[...])
    hr = jnp.maximum(comb, 0.0)
    mean = jnp.mean(hr, axis=0, keepdims=True)
    var = jnp.mean((hr - mean) * (hr - mean), axis=0, keepdims=True)
    hn = (hr - mean) * lax.rsqrt(var + _EPS) * g_ref[...] + b_ref[...]
    if final:
        out_refs[0][...] = (jnp.dot(hn, lw_ref[...],
                                    preferred_element_type=jnp.float32)
                            + lb_ref[...])
    else:
        out_refs[0][...] = hn[:, :_H]
        out_refs[1][...] = hn[:, _H:]


def _tc_layer(final, ha, hb, aggr, batch_col,
              vw, vb, aw, ab, rw, rb, g, b, lw, lb):
    if final:
        out_shape = jax.ShapeDtypeStruct((_N, lw.shape[1]), jnp.float32)
    else:
        out_shape = (jax.ShapeDtypeStruct((_N, _H), jnp.float32),
                     jax.ShapeDtypeStruct((_N, _H), jnp.float32))
    return pl.pallas_call(
        functools.partial(_tc_layer_body, final),
        out_shape=out_shape,
    )(ha, hb, aggr, batch_col, vw, vb.reshape(1, -1), aw, ab.reshape(1, -1),
      rw, rb.reshape(1, -1), g.reshape(1, -1), b.reshape(1, -1),
      lw, lb.reshape(1, -1))


def kernel(x, edge_index, batch,
           V0w, V0b, A0w, A0b, R0w, R0b, bn0_g, bn0_b,
           V1w, V1b, A1w, A1b, R1w, R1b, bn1_g, bn1_b,
           lin_w, lin_b):
    edges = jnp.stack([edge_index[0].reshape(_NSB, _SBE),
                       edge_index[1].reshape(_NSB, _SBE)],
                      axis=1)
    zeros = jnp.zeros((_NP, _H), jnp.float32)
    batch_col = batch.reshape(_N, 1)
    xa = x[:, :_H]
    xb = x[:, _H:]

    sc_scatter = _get_sc_scatter()
    aggr0 = sc_scatter(xa, xb, edges, zeros)
    h1a, h1b = _tc_layer(False, xa, xb, aggr0, batch_col,
                         V0w, V0b, A0w, A0b, R0w, R0b, bn0_g, bn0_b,
                         lin_w, lin_b)
    aggr1 = sc_scatter(h1a, h1b, edges, zeros)
    out = _tc_layer(True, h1a, h1b, aggr1, batch_col,
                    V1w, V1b, A1w, A1b, R1w, R1b, bn1_g, bn1_b, lin_w, lin_b)
    return out
